# Initial kernel scaffold; baseline (speedup 1.0000x reference)
import jax
import jax.numpy as jnp
from jax.experimental import pallas as pl


def kernel(inp, edgidx, h, Wxz, Whz, Wxr, Whr, Wxh, Whh, bxz, bhz, bxr, bhr, bxh, bhh):
    raise NotImplementedError("write your pallas kernel here")


# speedup vs baseline: 14.5388x; 14.5388x over previous
import functools

import jax
import jax.numpy as jnp
from jax import lax
from jax.experimental import pallas as pl
from jax.experimental.pallas import tpu as pltpu
from jax.experimental.pallas import tpu_sc as plsc

NC = 2
NS = 16
NW = NC * NS
LANES = 16

_mesh = functools.partial(
    plsc.VectorSubcoreMesh,
    core_axis_name="c", subcore_axis_name="s",
    num_cores=NC, num_subcores=NS,
)
_sc_params = pltpu.CompilerParams(needs_layout_passes=False)


def _make_deg(NPAD, E):
    RPT = NPAD // NS
    EPT = E // NW
    assert E % NW == 0 and EPT % LANES == 0 and RPT % LANES == 0

    @functools.partial(
        pl.kernel,
        out_type=jax.ShapeDtypeStruct((NC, NPAD), jnp.float32),
        mesh=_mesh(),
        scratch_types=[
            pltpu.VMEM((NPAD,), jnp.float32),
            pltpu.VMEM((EPT,), jnp.int32),
            pltpu.VMEM((RPT,), jnp.float32),
            pltpu.VMEM((RPT,), jnp.float32),
            pltpu.VMEM_SHARED((NS, NPAD), jnp.float32),
        ],
        compiler_params=_sc_params,
    )
    def deg_kernel(dst_hbm, out_hbm, hist, dbuf, acc, tmp, hist_all):
        c = lax.axis_index("c")
        s = lax.axis_index("s")
        wid = s * NC + c

        zero16 = jnp.zeros((LANES,), jnp.float32)

        def zero_body(j, _):
            hist[pl.ds(j * LANES, LANES)] = zero16
            return 0
        lax.fori_loop(0, NPAD // LANES, zero_body, 0)

        pltpu.sync_copy(dst_hbm.at[pl.ds(wid * EPT, EPT)], dbuf)

        ones16 = jnp.ones((LANES,), jnp.float32)

        def add_body(j, _):
            idx = dbuf[pl.ds(j * LANES, LANES)]
            plsc.addupdate_scatter(hist, [idx], ones16)
            return 0
        lax.fori_loop(0, EPT // LANES, add_body, 0)

        pltpu.sync_copy(hist, hist_all.at[s])
        plsc.subcore_barrier()

        base = s * RPT
        pltpu.sync_copy(hist_all.at[0, pl.ds(base, RPT)], acc)

        def comb_body(t, _):
            pltpu.sync_copy(hist_all.at[t, pl.ds(base, RPT)], tmp)

            def add16(j, _):
                sl = pl.ds(j * LANES, LANES)
                acc[sl] = acc[sl] + tmp[sl]
                return 0
            lax.fori_loop(0, RPT // LANES, add16, 0)
            return 0
        lax.fori_loop(1, NS, comb_body, 0)

        pltpu.sync_copy(acc, out_hbm.at[c, pl.ds(base, RPT)])

    return deg_kernel


def _make_prop(NPAD, E, D, dual):
    RPT = NPAD // NS
    K = 80
    EPT = E // NS if dual else E // NW
    assert EPT % K == 0 and RPT % 8 == 0
    nchunks = EPT // K

    @functools.partial(
        pl.kernel,
        out_type=jax.ShapeDtypeStruct((NC, NPAD, D), jnp.float32),
        mesh=_mesh(),
        scratch_types=[
            pltpu.VMEM((K,), jnp.int32),
            pltpu.VMEM((K,), jnp.int32),
            pltpu.VMEM((K, D), jnp.float32),
            pltpu.VMEM_SHARED((NPAD, D), jnp.float32),
            pltpu.SemaphoreType.DMA,
        ],
        compiler_params=_sc_params,
    )
    def prop_kernel(ya_hbm, yb_hbm, src_hbm, dst_hbm, out_hbm,
                    sidx, didx, rows, acc, sem):
        c = lax.axis_index("c")
        s = lax.axis_index("s")
        rbase = s * RPT

        @pl.when(c == 0)
        def _():
            pltpu.sync_copy(ya_hbm.at[pl.ds(rbase, RPT)],
                            acc.at[pl.ds(rbase, RPT)])

        @pl.when(c == 1)
        def _():
            pltpu.sync_copy(yb_hbm.at[pl.ds(rbase, RPT)],
                            acc.at[pl.ds(rbase, RPT)])

        plsc.subcore_barrier()

        if dual:
            ebase = s * EPT
        else:
            ebase = (s * NC + c) * EPT

        def body(i, _):
            eb = ebase + i * K
            pltpu.sync_copy(src_hbm.at[pl.ds(eb, K)], sidx)
            pltpu.sync_copy(dst_hbm.at[pl.ds(eb, K)], didx)
            if dual:
                @pl.when(c == 0)
                def _():
                    pltpu.async_copy(ya_hbm.at[sidx], rows, sem).wait()

                @pl.when(c == 1)
                def _():
                    pltpu.async_copy(yb_hbm.at[sidx], rows, sem).wait()
            else:
                pltpu.async_copy(ya_hbm.at[sidx], rows, sem).wait()
            pltpu.sync_copy(rows, acc.at[didx], add=True)
            return 0
        lax.fori_loop(0, nchunks, body, 0)

        plsc.subcore_barrier()
        pltpu.sync_copy(acc.at[pl.ds(rbase, RPT)],
                        out_hbm.at[c, pl.ds(rbase, RPT)])

    return prop_kernel


_R = 2048


def _dinv_of(deg_blk):
    return lax.rsqrt(deg_blk[:, 0:1] + deg_blk[:, 1:2] + 1.0)


def _t1(x, hi, degT, wxz, whz, wxr, whr):
    NP, D = x.shape

    def body(x_ref, h_ref, deg_ref, wxz_ref, whz_ref, wxr_ref, whr_ref,
             ya_ref, yb_ref):
        dinv = _dinv_of(deg_ref[...])
        xb = x_ref[...]
        hb = h_ref[...]
        ya_ref[...] = dinv * (
            jnp.dot(xb, wxz_ref[...], preferred_element_type=jnp.float32)
            + jnp.dot(hb, whz_ref[...], preferred_element_type=jnp.float32))
        yb_ref[...] = dinv * (
            jnp.dot(xb, wxr_ref[...], preferred_element_type=jnp.float32)
            + jnp.dot(hb, whr_ref[...], preferred_element_type=jnp.float32))

    row = pl.BlockSpec((_R, D), lambda i: (i, 0))
    w = pl.BlockSpec((D, D), lambda i: (0, 0))
    return pl.pallas_call(
        body,
        grid=(NP // _R,),
        in_specs=[row, row, pl.BlockSpec((_R, 2), lambda i: (i, 0)),
                  w, w, w, w],
        out_specs=[row, row],
        out_shape=[jax.ShapeDtypeStruct((NP, D), jnp.float32)] * 2,
    )(x, hi, degT, wxz, whz, wxr, whr)


def _t2(sa, sb, degT, x, hi, wxh, whh, bz, br):
    NP, D = x.shape

    def body(sa_ref, sb_ref, deg_ref, x_ref, h_ref, wxh_ref, whh_ref,
             bz_ref, br_ref, z_ref, y2_ref):
        dinv = _dinv_of(deg_ref[...])
        z = jax.nn.sigmoid(dinv * sa_ref[...] + bz_ref[...])
        r = jax.nn.sigmoid(dinv * sb_ref[...] + br_ref[...])
        y2 = dinv * (
            jnp.dot(x_ref[...], wxh_ref[...],
                    preferred_element_type=jnp.float32)
            + jnp.dot(r * h_ref[...], whh_ref[...],
                      preferred_element_type=jnp.float32))
        z_ref[...] = z
        y2_ref[...] = y2

    row = pl.BlockSpec((_R, D), lambda i: (i, 0))
    w = pl.BlockSpec((D, D), lambda i: (0, 0))
    b = pl.BlockSpec((1, D), lambda i: (0, 0))
    return pl.pallas_call(
        body,
        grid=(NP // _R,),
        in_specs=[row, row, pl.BlockSpec((_R, 2), lambda i: (i, 0)),
                  row, row, w, w, b, b],
        out_specs=[row, row],
        out_shape=[jax.ShapeDtypeStruct((NP, D), jnp.float32)] * 2,
    )(sa, sb, degT, x, hi, wxh, whh, bz, br)


def _t3(p0, p1, degT, z, hi, bh):
    NP, D = z.shape

    def body(p0_ref, p1_ref, deg_ref, z_ref, h_ref, bh_ref, out_ref):
        dinv = _dinv_of(deg_ref[...])
        htil = jnp.tanh(dinv * (p0_ref[...] + p1_ref[...]) + bh_ref[...])
        zb = z_ref[...]
        out_ref[...] = zb * h_ref[...] + (1.0 - zb) * htil

    row = pl.BlockSpec((_R, D), lambda i: (i, 0))
    b = pl.BlockSpec((1, D), lambda i: (0, 0))
    return pl.pallas_call(
        body,
        grid=(NP // _R,),
        in_specs=[row, row, pl.BlockSpec((_R, 2), lambda i: (i, 0)),
                  row, row, b],
        out_specs=row,
        out_shape=jax.ShapeDtypeStruct((NP, D), jnp.float32),
    )(p0, p1, degT, z, hi, bh)


def kernel(inp, edgidx, h, Wxz, Whz, Wxr, Whr, Wxh, Whh,
           bxz, bhz, bxr, bhr, bxh, bhh):
    N, D = inp.shape
    E = edgidx.shape[1]
    L = h.shape[0]
    NPAD = ((N + NW * LANES - 1) // (NW * LANES)) * (NW * LANES)
    assert NPAD % _R == 0

    src = edgidx[0].astype(jnp.int32)
    dst = edgidx[1].astype(jnp.int32)

    pad_n = NPAD - N
    xpad = jnp.pad(inp, ((0, pad_n), (0, 0)))
    hpad = jnp.pad(h, ((0, 0), (0, pad_n), (0, 0)))

    degp = _make_deg(NPAD, E)(dst)
    degT = jnp.transpose(degp)

    prop_dual = _make_prop(NPAD, E, D, dual=True)
    prop_split = _make_prop(NPAD, E, D, dual=False)

    zeros = jnp.zeros((NPAD, D), jnp.float32)

    x = xpad
    hs = []
    for i in range(L):
        hi = hpad[i]
        bz = (bxz[i] + bhz[i]).reshape(1, D)
        br = (bxr[i] + bhr[i]).reshape(1, D)
        bh = (bxh[i] + bhh[i]).reshape(1, D)

        ya, yb = _t1(x, hi, degT, Wxz[i], Whz[i], Wxr[i], Whr[i])
        s_ab = prop_dual(ya, yb, src, dst)
        z, y2 = _t2(s_ab[0], s_ab[1], degT, x, hi, Wxh[i], Whh[i], bz, br)
        p = prop_split(y2, zeros, src, dst)
        x = _t3(p[0], p[1], degT, z, hi, bh)
        hs.append(x)

    h_out = jnp.stack(hs, axis=0)[:, :N, :]
    return (h_out, h_out)


# speedup vs baseline: 30.0423x; 2.0664x over previous
import functools

import jax
import jax.numpy as jnp
from jax import lax
from jax.experimental import pallas as pl
from jax.experimental.pallas import tpu as pltpu
from jax.experimental.pallas import tpu_sc as plsc

NC = 2
NS = 16
NW = NC * NS
LANES = 16

_mesh = functools.partial(
    plsc.VectorSubcoreMesh,
    core_axis_name="c", subcore_axis_name="s",
    num_cores=NC, num_subcores=NS,
)
_sc_params = pltpu.CompilerParams(needs_layout_passes=False)


def _make_deg(NPAD, E):
    RPT = NPAD // NS
    EPT = E // NW
    assert E % NW == 0 and EPT % LANES == 0 and RPT % LANES == 0

    @functools.partial(
        pl.kernel,
        out_type=jax.ShapeDtypeStruct((NC, NPAD), jnp.float32),
        mesh=_mesh(),
        scratch_types=[
            pltpu.VMEM((NPAD,), jnp.float32),
            pltpu.VMEM((EPT,), jnp.int32),
            pltpu.VMEM((RPT,), jnp.float32),
            pltpu.VMEM((RPT,), jnp.float32),
            pltpu.VMEM_SHARED((NS, NPAD), jnp.float32),
        ],
        compiler_params=_sc_params,
    )
    def deg_kernel(dst_hbm, out_hbm, hist, dbuf, acc, tmp, hist_all):
        c = lax.axis_index("c")
        s = lax.axis_index("s")
        wid = s * NC + c

        zero16 = jnp.zeros((LANES,), jnp.float32)

        def zero_body(j, _):
            hist[pl.ds(j * LANES, LANES)] = zero16
            return 0
        lax.fori_loop(0, NPAD // LANES, zero_body, 0)

        pltpu.sync_copy(dst_hbm.at[pl.ds(wid * EPT, EPT)], dbuf)

        ones16 = jnp.ones((LANES,), jnp.float32)

        def add_body(j, _):
            idx = dbuf[pl.ds(j * LANES, LANES)]
            plsc.addupdate_scatter(hist, [idx], ones16)
            return 0
        lax.fori_loop(0, EPT // LANES, add_body, 0)

        pltpu.sync_copy(hist, hist_all.at[s])
        plsc.subcore_barrier()

        base = s * RPT
        pltpu.sync_copy(hist_all.at[0, pl.ds(base, RPT)], acc)

        def comb_body(t, _):
            pltpu.sync_copy(hist_all.at[t, pl.ds(base, RPT)], tmp)

            def add16(j, _):
                sl = pl.ds(j * LANES, LANES)
                acc[sl] = acc[sl] + tmp[sl]
                return 0
            lax.fori_loop(0, RPT // LANES, add16, 0)
            return 0
        lax.fori_loop(1, NS, comb_body, 0)

        pltpu.sync_copy(acc, out_hbm.at[c, pl.ds(base, RPT)])

    return deg_kernel


_K = 125


def _make_prop(NPAD, E, D, dual):
    K = _K
    RPT = NPAD // NS
    EPT = E // NS if dual else E // NW
    assert EPT % K == 0 and RPT % 8 == 0
    nch = EPT // K
    IB = 16
    assert nch % IB == 0
    nblk = nch // IB


    @functools.partial(
        pl.kernel,
        out_type=jax.ShapeDtypeStruct((NC, NPAD, D), jnp.float32),
        mesh=_mesh(),
        scratch_types=[
            pltpu.VMEM((IB, K), jnp.int32),
            pltpu.VMEM((IB, K), jnp.int32),
            [pltpu.VMEM((K, D), jnp.float32)] * 2,
            [pltpu.SemaphoreType.DMA] * 2,
            [pltpu.SemaphoreType.DMA] * 2,
            pltpu.VMEM_SHARED((NPAD, D), jnp.float32),
        ],
        compiler_params=_sc_params,
    )
    def prop_kernel(ya_hbm, yb_hbm, src_hbm, dst_hbm, out_hbm,
                    sblk, dblk, rows, gsem, ssem, acc):
        c = lax.axis_index("c")
        s = lax.axis_index("s")
        rbase = s * RPT

        @pl.when(c == 0)
        def _():
            pltpu.sync_copy(ya_hbm.at[pl.ds(rbase, RPT)],
                            acc.at[pl.ds(rbase, RPT)])

        @pl.when(c == 1)
        def _():
            pltpu.sync_copy(yb_hbm.at[pl.ds(rbase, RPT)],
                            acc.at[pl.ds(rbase, RPT)])

        cbase = (s if dual else s * NC + c) * nch
        plsc.subcore_barrier()

        def issue_g(j, b):
            if dual:
                @pl.when(c == 0)
                def _():
                    pltpu.async_copy(ya_hbm.at[sblk.at[j]], rows[b], gsem[b])

                @pl.when(c == 1)
                def _():
                    pltpu.async_copy(yb_hbm.at[sblk.at[j]], rows[b], gsem[b])
            else:
                pltpu.async_copy(ya_hbm.at[sblk.at[j]], rows[b], gsem[b])

        def wait_g(j, b):
            pltpu.make_async_copy(ya_hbm.at[sblk.at[j]],
                                  rows[b], gsem[b]).wait()

        def issue_s(j, b):
            pltpu.async_copy(rows[b], acc.at[dblk.at[j]], ssem[b], add=True)

        def wait_s(j, b):
            pltpu.make_async_copy(rows[b], acc.at[dblk.at[j]],
                                  ssem[b]).wait()

        def blk_body(blk, _):
            row0 = cbase + blk * IB
            pltpu.sync_copy(src_hbm.at[pl.ds(row0, IB)], sblk)
            pltpu.sync_copy(dst_hbm.at[pl.ds(row0, IB)], dblk)
            issue_g(0, 0)

            def pair_body(q, _):
                j = 2 * q
                wait_g(j, 0)
                issue_s(j, 0)

                @pl.when(q > 0)
                def _():
                    wait_s(j - 1, 1)
                issue_g(j + 1, 1)

                wait_g(j + 1, 1)
                issue_s(j + 1, 1)
                wait_s(j, 0)

                @pl.when(q < IB // 2 - 1)
                def _():
                    issue_g(j + 2, 0)
                return 0
            lax.fori_loop(0, IB // 2, pair_body, 0)
            wait_s(IB - 1, 1)
            return 0
        lax.fori_loop(0, nblk, blk_body, 0)

        plsc.subcore_barrier()
        pltpu.sync_copy(acc.at[pl.ds(rbase, RPT)],
                        out_hbm.at[c, pl.ds(rbase, RPT)])

    return prop_kernel


_R = 2048


def _dinv_of(deg_blk):
    return lax.rsqrt(deg_blk[:, 0:1] + deg_blk[:, 1:2] + 1.0)


def _t1(x, hi, degT, wxz, whz, wxr, whr):
    NP, D = x.shape

    def body(x_ref, h_ref, deg_ref, wxz_ref, whz_ref, wxr_ref, whr_ref,
             ya_ref, yb_ref):
        dinv = _dinv_of(deg_ref[...])
        xb = x_ref[...]
        hb = h_ref[...]
        ya_ref[...] = dinv * (
            jnp.dot(xb, wxz_ref[...], preferred_element_type=jnp.float32)
            + jnp.dot(hb, whz_ref[...], preferred_element_type=jnp.float32))
        yb_ref[...] = dinv * (
            jnp.dot(xb, wxr_ref[...], preferred_element_type=jnp.float32)
            + jnp.dot(hb, whr_ref[...], preferred_element_type=jnp.float32))

    row = pl.BlockSpec((_R, D), lambda i: (i, 0))
    w = pl.BlockSpec((D, D), lambda i: (0, 0))
    return pl.pallas_call(
        body,
        grid=(NP // _R,),
        in_specs=[row, row, pl.BlockSpec((_R, 2), lambda i: (i, 0)),
                  w, w, w, w],
        out_specs=[row, row],
        out_shape=[jax.ShapeDtypeStruct((NP, D), jnp.float32)] * 2,
    )(x, hi, degT, wxz, whz, wxr, whr)


def _t2(sa, sb, degT, x, hi, wxh, whh, bz, br):
    NP, D = x.shape

    def body(sa_ref, sb_ref, deg_ref, x_ref, h_ref, wxh_ref, whh_ref,
             bz_ref, br_ref, z_ref, y2_ref):
        dinv = _dinv_of(deg_ref[...])
        z = jax.nn.sigmoid(dinv * sa_ref[...] + bz_ref[...])
        r = jax.nn.sigmoid(dinv * sb_ref[...] + br_ref[...])
        y2 = dinv * (
            jnp.dot(x_ref[...], wxh_ref[...],
                    preferred_element_type=jnp.float32)
            + jnp.dot(r * h_ref[...], whh_ref[...],
                      preferred_element_type=jnp.float32))
        z_ref[...] = z
        y2_ref[...] = y2

    row = pl.BlockSpec((_R, D), lambda i: (i, 0))
    w = pl.BlockSpec((D, D), lambda i: (0, 0))
    b = pl.BlockSpec((1, D), lambda i: (0, 0))
    return pl.pallas_call(
        body,
        grid=(NP // _R,),
        in_specs=[row, row, pl.BlockSpec((_R, 2), lambda i: (i, 0)),
                  row, row, w, w, b, b],
        out_specs=[row, row],
        out_shape=[jax.ShapeDtypeStruct((NP, D), jnp.float32)] * 2,
    )(sa, sb, degT, x, hi, wxh, whh, bz, br)


def _t3(p0, p1, degT, z, hi, bh):
    NP, D = z.shape

    def body(p0_ref, p1_ref, deg_ref, z_ref, h_ref, bh_ref, out_ref):
        dinv = _dinv_of(deg_ref[...])
        htil = jnp.tanh(dinv * (p0_ref[...] + p1_ref[...]) + bh_ref[...])
        zb = z_ref[...]
        out_ref[...] = zb * h_ref[...] + (1.0 - zb) * htil

    row = pl.BlockSpec((_R, D), lambda i: (i, 0))
    b = pl.BlockSpec((1, D), lambda i: (0, 0))
    return pl.pallas_call(
        body,
        grid=(NP // _R,),
        in_specs=[row, row, pl.BlockSpec((_R, 2), lambda i: (i, 0)),
                  row, row, b],
        out_specs=row,
        out_shape=jax.ShapeDtypeStruct((NP, D), jnp.float32),
    )(p0, p1, degT, z, hi, bh)


def kernel(inp, edgidx, h, Wxz, Whz, Wxr, Whr, Wxh, Whh,
           bxz, bhz, bxr, bhr, bxh, bhh):
    N, D = inp.shape
    E = edgidx.shape[1]
    L = h.shape[0]
    NPAD = ((N + NW * LANES - 1) // (NW * LANES)) * (NW * LANES)
    assert NPAD % _R == 0

    assert E % _K == 0
    src = edgidx[0].astype(jnp.int32)
    dst = edgidx[1].astype(jnp.int32)
    src2d = src.reshape(E // _K, _K)
    dst2d = dst.reshape(E // _K, _K)

    pad_n = NPAD - N
    xpad = jnp.pad(inp, ((0, pad_n), (0, 0)))
    hpad = jnp.pad(h, ((0, 0), (0, pad_n), (0, 0)))

    degp = _make_deg(NPAD, E)(dst)
    degT = jnp.transpose(degp)

    prop_dual = _make_prop(NPAD, E, D, dual=True)
    prop_split = _make_prop(NPAD, E, D, dual=False)

    zeros = jnp.zeros((NPAD, D), jnp.float32)

    x = xpad
    hs = []
    for i in range(L):
        hi = hpad[i]
        bz = (bxz[i] + bhz[i]).reshape(1, D)
        br = (bxr[i] + bhr[i]).reshape(1, D)
        bh = (bxh[i] + bhh[i]).reshape(1, D)

        ya, yb = _t1(x, hi, degT, Wxz[i], Whz[i], Wxr[i], Whr[i])
        s_ab = prop_dual(ya, yb, src2d, dst2d)
        z, y2 = _t2(s_ab[0], s_ab[1], degT, x, hi, Wxh[i], Whh[i], bz, br)
        p = prop_split(y2, zeros, src2d, dst2d)
        x = _t3(p[0], p[1], degT, z, hi, bh)
        hs.append(x)

    h_out = jnp.stack(hs, axis=0)[:, :N, :]
    return (h_out, h_out)


# speedup vs baseline: 30.8286x; 1.0262x over previous
import functools

import jax
import jax.numpy as jnp
from jax import lax
from jax.experimental import pallas as pl
from jax.experimental.pallas import tpu as pltpu
from jax.experimental.pallas import tpu_sc as plsc

NC = 2
NS = 16
NW = NC * NS
LANES = 16

_mesh = functools.partial(
    plsc.VectorSubcoreMesh,
    core_axis_name="c", subcore_axis_name="s",
    num_cores=NC, num_subcores=NS,
)
_sc_params = pltpu.CompilerParams(needs_layout_passes=False)


def _make_deg(NPAD, E):
    RPT = NPAD // NS
    EPT = E // NW
    assert E % NW == 0 and EPT % LANES == 0 and RPT % LANES == 0

    @functools.partial(
        pl.kernel,
        out_type=jax.ShapeDtypeStruct((NC, NPAD), jnp.float32),
        mesh=_mesh(),
        scratch_types=[
            pltpu.VMEM((NPAD,), jnp.float32),
            pltpu.VMEM((EPT,), jnp.int32),
            pltpu.VMEM((RPT,), jnp.float32),
            pltpu.VMEM((RPT,), jnp.float32),
            pltpu.VMEM_SHARED((NS, NPAD), jnp.float32),
        ],
        compiler_params=_sc_params,
    )
    def deg_kernel(dst_hbm, out_hbm, hist, dbuf, acc, tmp, hist_all):
        c = lax.axis_index("c")
        s = lax.axis_index("s")
        wid = s * NC + c

        zero16 = jnp.zeros((LANES,), jnp.float32)

        def zero_body(j, _):
            hist[pl.ds(j * LANES, LANES)] = zero16
            return 0
        lax.fori_loop(0, NPAD // LANES, zero_body, 0)

        pltpu.sync_copy(dst_hbm.at[pl.ds(wid * EPT, EPT)], dbuf)

        ones16 = jnp.ones((LANES,), jnp.float32)

        def add_body(j, _):
            idx = dbuf[pl.ds(j * LANES, LANES)]
            plsc.addupdate_scatter(hist, [idx], ones16)
            return 0
        lax.fori_loop(0, EPT // LANES, add_body, 0)

        pltpu.sync_copy(hist, hist_all.at[s])
        plsc.subcore_barrier()

        base = s * RPT
        pltpu.sync_copy(hist_all.at[0, pl.ds(base, RPT)], acc)

        def comb_body(t, _):
            pltpu.sync_copy(hist_all.at[t, pl.ds(base, RPT)], tmp)

            def add16(j, _):
                sl = pl.ds(j * LANES, LANES)
                acc[sl] = acc[sl] + tmp[sl]
                return 0
            lax.fori_loop(0, RPT // LANES, add16, 0)
            return 0
        lax.fori_loop(1, NS, comb_body, 0)

        pltpu.sync_copy(acc, out_hbm.at[c, pl.ds(base, RPT)])

    return deg_kernel


_K = 125


def _make_prop(NPAD, E, D, dual):
    K = _K
    RPT = NPAD // NS
    EPT = E // NS if dual else E // NW
    assert EPT % K == 0 and RPT % 8 == 0
    nch = EPT // K
    IB = 32 if dual else 16
    assert nch % IB == 0
    nblk = nch // IB


    @functools.partial(
        pl.kernel,
        out_type=jax.ShapeDtypeStruct((NC, NPAD, D), jnp.float32),
        mesh=_mesh(),
        scratch_types=[
            pltpu.VMEM((IB, K), jnp.int32),
            pltpu.VMEM((IB, K), jnp.int32),
            [pltpu.VMEM((K, D), jnp.float32)] * 2,
            [pltpu.SemaphoreType.DMA] * 2,
            [pltpu.SemaphoreType.DMA] * 2,
            pltpu.VMEM_SHARED((NPAD, D), jnp.float32),
        ],
        compiler_params=_sc_params,
    )
    def prop_kernel(ya_hbm, yb_hbm, src_hbm, dst_hbm, out_hbm,
                    sblk, dblk, rows, gsem, ssem, acc):
        c = lax.axis_index("c")
        s = lax.axis_index("s")
        rbase = s * RPT

        @pl.when(c == 0)
        def _():
            pltpu.sync_copy(ya_hbm.at[pl.ds(rbase, RPT)],
                            acc.at[pl.ds(rbase, RPT)])

        @pl.when(c == 1)
        def _():
            pltpu.sync_copy(yb_hbm.at[pl.ds(rbase, RPT)],
                            acc.at[pl.ds(rbase, RPT)])

        cbase = (s if dual else s * NC + c) * nch
        plsc.subcore_barrier()

        def issue_g(j, b):
            if dual:
                @pl.when(c == 0)
                def _():
                    pltpu.async_copy(ya_hbm.at[sblk.at[j]], rows[b], gsem[b])

                @pl.when(c == 1)
                def _():
                    pltpu.async_copy(yb_hbm.at[sblk.at[j]], rows[b], gsem[b])
            else:
                pltpu.async_copy(ya_hbm.at[sblk.at[j]], rows[b], gsem[b])

        def wait_g(j, b):
            pltpu.make_async_copy(ya_hbm.at[sblk.at[j]],
                                  rows[b], gsem[b]).wait()

        def issue_s(j, b):
            pltpu.async_copy(rows[b], acc.at[dblk.at[j]], ssem[b], add=True)

        def wait_s(j, b):
            pltpu.make_async_copy(rows[b], acc.at[dblk.at[j]],
                                  ssem[b]).wait()

        def blk_body(blk, _):
            row0 = cbase + blk * IB
            pltpu.sync_copy(src_hbm.at[pl.ds(row0, IB)], sblk)
            pltpu.sync_copy(dst_hbm.at[pl.ds(row0, IB)], dblk)
            issue_g(0, 0)

            def pair_body(q, _):
                j = 2 * q
                wait_g(j, 0)
                issue_s(j, 0)

                @pl.when(q > 0)
                def _():
                    wait_s(j - 1, 1)
                issue_g(j + 1, 1)

                wait_g(j + 1, 1)
                issue_s(j + 1, 1)
                wait_s(j, 0)

                @pl.when(q < IB // 2 - 1)
                def _():
                    issue_g(j + 2, 0)
                return 0
            lax.fori_loop(0, IB // 2, pair_body, 0)
            wait_s(IB - 1, 1)
            return 0
        lax.fori_loop(0, nblk, blk_body, 0)

        plsc.subcore_barrier()
        pltpu.sync_copy(acc.at[pl.ds(rbase, RPT)],
                        out_hbm.at[c, pl.ds(rbase, RPT)])

    return prop_kernel


_R = 2048


def _dinv_of(deg_blk):
    return lax.rsqrt(deg_blk[:, 0:1] + deg_blk[:, 1:2] + 1.0)


def _t1(x, hi, degT, wxz, whz, wxr, whr):
    NP, D = x.shape

    def body(x_ref, h_ref, deg_ref, wxz_ref, whz_ref, wxr_ref, whr_ref,
             ya_ref, yb_ref):
        dinv = _dinv_of(deg_ref[...])
        xb = x_ref[...]
        hb = h_ref[...]
        ya_ref[...] = dinv * (
            jnp.dot(xb, wxz_ref[...], preferred_element_type=jnp.float32)
            + jnp.dot(hb, whz_ref[...], preferred_element_type=jnp.float32))
        yb_ref[...] = dinv * (
            jnp.dot(xb, wxr_ref[...], preferred_element_type=jnp.float32)
            + jnp.dot(hb, whr_ref[...], preferred_element_type=jnp.float32))

    row = pl.BlockSpec((_R, D), lambda i: (i, 0))
    w = pl.BlockSpec((D, D), lambda i: (0, 0))
    return pl.pallas_call(
        body,
        grid=(NP // _R,),
        in_specs=[row, row, pl.BlockSpec((_R, 2), lambda i: (i, 0)),
                  w, w, w, w],
        out_specs=[row, row],
        out_shape=[jax.ShapeDtypeStruct((NP, D), jnp.float32)] * 2,
    )(x, hi, degT, wxz, whz, wxr, whr)


def _t2(sa, sb, degT, x, hi, wxh, whh, bz, br):
    NP, D = x.shape

    def body(sa_ref, sb_ref, deg_ref, x_ref, h_ref, wxh_ref, whh_ref,
             bz_ref, br_ref, z_ref, y2_ref):
        dinv = _dinv_of(deg_ref[...])
        z = jax.nn.sigmoid(dinv * sa_ref[...] + bz_ref[...])
        r = jax.nn.sigmoid(dinv * sb_ref[...] + br_ref[...])
        y2 = dinv * (
            jnp.dot(x_ref[...], wxh_ref[...],
                    preferred_element_type=jnp.float32)
            + jnp.dot(r * h_ref[...], whh_ref[...],
                      preferred_element_type=jnp.float32))
        z_ref[...] = z
        y2_ref[...] = y2

    row = pl.BlockSpec((_R, D), lambda i: (i, 0))
    w = pl.BlockSpec((D, D), lambda i: (0, 0))
    b = pl.BlockSpec((1, D), lambda i: (0, 0))
    return pl.pallas_call(
        body,
        grid=(NP // _R,),
        in_specs=[row, row, pl.BlockSpec((_R, 2), lambda i: (i, 0)),
                  row, row, w, w, b, b],
        out_specs=[row, row],
        out_shape=[jax.ShapeDtypeStruct((NP, D), jnp.float32)] * 2,
    )(sa, sb, degT, x, hi, wxh, whh, bz, br)


def _t31(p0, p1, degT, z, hi, bh, hnext, wxz, whz, wxr, whr):
    NP, D = z.shape

    def body(p0_ref, p1_ref, deg_ref, z_ref, h_ref, bh_ref, hn_ref,
             wxz_ref, whz_ref, wxr_ref, whr_ref,
             out_ref, ya_ref, yb_ref):
        dinv = _dinv_of(deg_ref[...])
        htil = jnp.tanh(dinv * (p0_ref[...] + p1_ref[...]) + bh_ref[...])
        zb = z_ref[...]
        xb = zb * h_ref[...] + (1.0 - zb) * htil
        out_ref[...] = xb
        hb = hn_ref[...]
        ya_ref[...] = dinv * (
            jnp.dot(xb, wxz_ref[...], preferred_element_type=jnp.float32)
            + jnp.dot(hb, whz_ref[...], preferred_element_type=jnp.float32))
        yb_ref[...] = dinv * (
            jnp.dot(xb, wxr_ref[...], preferred_element_type=jnp.float32)
            + jnp.dot(hb, whr_ref[...], preferred_element_type=jnp.float32))

    row = pl.BlockSpec((_R, D), lambda i: (i, 0))
    w = pl.BlockSpec((D, D), lambda i: (0, 0))
    b = pl.BlockSpec((1, D), lambda i: (0, 0))
    return pl.pallas_call(
        body,
        grid=(NP // _R,),
        in_specs=[row, row, pl.BlockSpec((_R, 2), lambda i: (i, 0)),
                  row, row, b, row, w, w, w, w],
        out_specs=[row, row, row],
        out_shape=[jax.ShapeDtypeStruct((NP, D), jnp.float32)] * 3,
    )(p0, p1, degT, z, hi, bh, hnext, wxz, whz, wxr, whr)


def _t3(p0, p1, degT, z, hi, bh):
    NP, D = z.shape

    def body(p0_ref, p1_ref, deg_ref, z_ref, h_ref, bh_ref, out_ref):
        dinv = _dinv_of(deg_ref[...])
        htil = jnp.tanh(dinv * (p0_ref[...] + p1_ref[...]) + bh_ref[...])
        zb = z_ref[...]
        out_ref[...] = zb * h_ref[...] + (1.0 - zb) * htil

    row = pl.BlockSpec((_R, D), lambda i: (i, 0))
    b = pl.BlockSpec((1, D), lambda i: (0, 0))
    return pl.pallas_call(
        body,
        grid=(NP // _R,),
        in_specs=[row, row, pl.BlockSpec((_R, 2), lambda i: (i, 0)),
                  row, row, b],
        out_specs=row,
        out_shape=jax.ShapeDtypeStruct((NP, D), jnp.float32),
    )(p0, p1, degT, z, hi, bh)


def kernel(inp, edgidx, h, Wxz, Whz, Wxr, Whr, Wxh, Whh,
           bxz, bhz, bxr, bhr, bxh, bhh):
    N, D = inp.shape
    E = edgidx.shape[1]
    L = h.shape[0]
    NPAD = ((N + NW * LANES - 1) // (NW * LANES)) * (NW * LANES)
    assert NPAD % _R == 0

    assert E % _K == 0
    src = edgidx[0].astype(jnp.int32)
    dst = edgidx[1].astype(jnp.int32)
    src2d = src.reshape(E // _K, _K)
    dst2d = dst.reshape(E // _K, _K)

    pad_n = NPAD - N
    xpad = jnp.pad(inp, ((0, pad_n), (0, 0)))
    hpad = jnp.pad(h, ((0, 0), (0, pad_n), (0, 0)))

    degp = _make_deg(NPAD, E)(dst)
    degT = jnp.transpose(degp)

    prop_dual = _make_prop(NPAD, E, D, dual=True)
    prop_split = _make_prop(NPAD, E, D, dual=False)

    zeros = jnp.zeros((NPAD, D), jnp.float32)

    x = xpad
    hs = []
    ya, yb = _t1(x, hpad[0], degT, Wxz[0], Whz[0], Wxr[0], Whr[0])
    for i in range(L):
        hi = hpad[i]
        bz = (bxz[i] + bhz[i]).reshape(1, D)
        br = (bxr[i] + bhr[i]).reshape(1, D)
        bh = (bxh[i] + bhh[i]).reshape(1, D)

        s_ab = prop_dual(ya, yb, src2d, dst2d)
        z, y2 = _t2(s_ab[0], s_ab[1], degT, x, hi, Wxh[i], Whh[i], bz, br)
        p = prop_split(y2, zeros, src2d, dst2d)
        if i + 1 < L:
            x, ya, yb = _t31(p[0], p[1], degT, z, hi, bh, hpad[i + 1],
                             Wxz[i + 1], Whz[i + 1], Wxr[i + 1], Whr[i + 1])
        else:
            x = _t3(p[0], p[1], degT, z, hi, bh)
        hs.append(x)

    h_out = jnp.stack(hs, axis=0)[:, :N, :]
    return (h_out, h_out)


# speedup vs baseline: 31.8640x; 1.0336x over previous
import functools

import jax
import jax.numpy as jnp
from jax import lax
from jax.experimental import pallas as pl
from jax.experimental.pallas import tpu as pltpu
from jax.experimental.pallas import tpu_sc as plsc

NC = 2
NS = 16
NW = NC * NS
LANES = 16

_mesh = functools.partial(
    plsc.VectorSubcoreMesh,
    core_axis_name="c", subcore_axis_name="s",
    num_cores=NC, num_subcores=NS,
)
_sc_params = pltpu.CompilerParams(needs_layout_passes=False)


def _make_deg(NPAD, E):
    RPT = NPAD // NS
    EPT = E // NW
    assert E % NW == 0 and EPT % LANES == 0 and RPT % LANES == 0

    @functools.partial(
        pl.kernel,
        out_type=jax.ShapeDtypeStruct((NC, NPAD), jnp.float32),
        mesh=_mesh(),
        scratch_types=[
            pltpu.VMEM((NPAD,), jnp.float32),
            pltpu.VMEM((EPT,), jnp.int32),
            pltpu.VMEM((RPT,), jnp.float32),
            pltpu.VMEM((RPT,), jnp.float32),
            pltpu.VMEM_SHARED((NS, NPAD), jnp.float32),
        ],
        compiler_params=_sc_params,
    )
    def deg_kernel(dst_hbm, out_hbm, hist, dbuf, acc, tmp, hist_all):
        c = lax.axis_index("c")
        s = lax.axis_index("s")
        wid = s * NC + c

        zero16 = jnp.zeros((LANES,), jnp.float32)

        def zero_body(j, _):
            hist[pl.ds(j * LANES, LANES)] = zero16
            return 0
        lax.fori_loop(0, NPAD // LANES, zero_body, 0)

        pltpu.sync_copy(dst_hbm.at[pl.ds(wid * EPT, EPT)], dbuf)

        ones16 = jnp.ones((LANES,), jnp.float32)

        def add_body(j, _):
            idx = dbuf[pl.ds(j * LANES, LANES)]
            plsc.addupdate_scatter(hist, [idx], ones16)
            return 0
        lax.fori_loop(0, EPT // LANES, add_body, 0)

        pltpu.sync_copy(hist, hist_all.at[s])
        plsc.subcore_barrier()

        base = s * RPT
        pltpu.sync_copy(hist_all.at[0, pl.ds(base, RPT)], acc)

        def comb_body(t, _):
            pltpu.sync_copy(hist_all.at[t, pl.ds(base, RPT)], tmp)

            def add16(j, _):
                sl = pl.ds(j * LANES, LANES)
                acc[sl] = acc[sl] + tmp[sl]
                return 0
            lax.fori_loop(0, RPT // LANES, add16, 0)
            return 0
        lax.fori_loop(1, NS, comb_body, 0)

        pltpu.sync_copy(acc, out_hbm.at[c, pl.ds(base, RPT)])

    return deg_kernel


_K = 125


def _make_prop(NPAD, E, D, dual):
    K = _K
    RPT = NPAD // NS
    EPT = E // NS if dual else E // NW
    assert EPT % K == 0 and RPT % 8 == 0
    nch = EPT // K
    IB = 32 if dual else 16
    assert nch % IB == 0
    nblk = nch // IB


    @functools.partial(
        pl.kernel,
        out_type=[jax.ShapeDtypeStruct((NPAD, D), jnp.float32)] * 2,
        mesh=_mesh(),
        scratch_types=[
            pltpu.VMEM((IB, K), jnp.int32),
            pltpu.VMEM((IB, K), jnp.int32),
            [pltpu.VMEM((K, D), jnp.float32)] * 2,
            [pltpu.SemaphoreType.DMA] * 2,
            [pltpu.SemaphoreType.DMA] * 2,
            pltpu.VMEM_SHARED((NPAD, D), jnp.float32),
        ],
        compiler_params=_sc_params,
    )
    def prop_kernel(ya_hbm, yb_hbm, src_hbm, dst_hbm, outa_hbm, outb_hbm,
                    sblk, dblk, rows, gsem, ssem, acc):
        c = lax.axis_index("c")
        s = lax.axis_index("s")
        rbase = s * RPT

        @pl.when(c == 0)
        def _():
            pltpu.sync_copy(ya_hbm.at[pl.ds(rbase, RPT)],
                            acc.at[pl.ds(rbase, RPT)])

        @pl.when(c == 1)
        def _():
            pltpu.sync_copy(yb_hbm.at[pl.ds(rbase, RPT)],
                            acc.at[pl.ds(rbase, RPT)])

        cbase = (s if dual else s * NC + c) * nch
        plsc.subcore_barrier()

        def issue_g(j, b):
            if dual:
                @pl.when(c == 0)
                def _():
                    pltpu.async_copy(ya_hbm.at[sblk.at[j]], rows[b], gsem[b])

                @pl.when(c == 1)
                def _():
                    pltpu.async_copy(yb_hbm.at[sblk.at[j]], rows[b], gsem[b])
            else:
                pltpu.async_copy(ya_hbm.at[sblk.at[j]], rows[b], gsem[b])

        def wait_g(j, b):
            pltpu.make_async_copy(ya_hbm.at[sblk.at[j]],
                                  rows[b], gsem[b]).wait()

        def issue_s(j, b):
            pltpu.async_copy(rows[b], acc.at[dblk.at[j]], ssem[b], add=True)

        def wait_s(j, b):
            pltpu.make_async_copy(rows[b], acc.at[dblk.at[j]],
                                  ssem[b]).wait()

        def blk_body(blk, _):
            row0 = cbase + blk * IB
            pltpu.sync_copy(src_hbm.at[pl.ds(row0, IB)], sblk)
            pltpu.sync_copy(dst_hbm.at[pl.ds(row0, IB)], dblk)
            issue_g(0, 0)

            def pair_body(q, _):
                j = 2 * q
                wait_g(j, 0)
                issue_s(j, 0)

                @pl.when(q > 0)
                def _():
                    wait_s(j - 1, 1)
                issue_g(j + 1, 1)

                wait_g(j + 1, 1)
                issue_s(j + 1, 1)
                wait_s(j, 0)

                @pl.when(q < IB // 2 - 1)
                def _():
                    issue_g(j + 2, 0)
                return 0
            lax.fori_loop(0, IB // 2, pair_body, 0)
            wait_s(IB - 1, 1)
            return 0
        lax.fori_loop(0, nblk, blk_body, 0)

        plsc.subcore_barrier()

        @pl.when(c == 0)
        def _():
            pltpu.sync_copy(acc.at[pl.ds(rbase, RPT)],
                            outa_hbm.at[pl.ds(rbase, RPT)])

        @pl.when(c == 1)
        def _():
            pltpu.sync_copy(acc.at[pl.ds(rbase, RPT)],
                            outb_hbm.at[pl.ds(rbase, RPT)])

    return prop_kernel


_R = 2048


def _dinv_of(deg_blk):
    return lax.rsqrt(deg_blk[:, 0:1] + deg_blk[:, 1:2] + 1.0)


def _t1(x, hi, degT, wxz, whz, wxr, whr):
    NP, D = x.shape

    def body(x_ref, h_ref, deg_ref, wxz_ref, whz_ref, wxr_ref, whr_ref,
             ya_ref, yb_ref):
        dinv = _dinv_of(deg_ref[...])
        xb = x_ref[...]
        hb = h_ref[...]
        ya_ref[...] = dinv * (
            jnp.dot(xb, wxz_ref[...], preferred_element_type=jnp.float32)
            + jnp.dot(hb, whz_ref[...], preferred_element_type=jnp.float32))
        yb_ref[...] = dinv * (
            jnp.dot(xb, wxr_ref[...], preferred_element_type=jnp.float32)
            + jnp.dot(hb, whr_ref[...], preferred_element_type=jnp.float32))

    row = pl.BlockSpec((_R, D), lambda i: (i, 0))
    w = pl.BlockSpec((D, D), lambda i: (0, 0))
    return pl.pallas_call(
        body,
        grid=(NP // _R,),
        in_specs=[row, row, pl.BlockSpec((_R, 2), lambda i: (i, 0)),
                  w, w, w, w],
        out_specs=[row, row],
        out_shape=[jax.ShapeDtypeStruct((NP, D), jnp.float32)] * 2,
    )(x, hi, degT, wxz, whz, wxr, whr)


def _t2(sa, sb, degT, x, hi, wxh, whh, bz, br):
    NP, D = x.shape

    def body(sa_ref, sb_ref, deg_ref, x_ref, h_ref, wxh_ref, whh_ref,
             bz_ref, br_ref, z_ref, y2_ref):
        dinv = _dinv_of(deg_ref[...])
        z = jax.nn.sigmoid(dinv * sa_ref[...] + bz_ref[...])
        r = jax.nn.sigmoid(dinv * sb_ref[...] + br_ref[...])
        y2 = dinv * (
            jnp.dot(x_ref[...], wxh_ref[...],
                    preferred_element_type=jnp.float32)
            + jnp.dot(r * h_ref[...], whh_ref[...],
                      preferred_element_type=jnp.float32))
        z_ref[...] = z
        y2_ref[...] = y2

    row = pl.BlockSpec((_R, D), lambda i: (i, 0))
    w = pl.BlockSpec((D, D), lambda i: (0, 0))
    b = pl.BlockSpec((1, D), lambda i: (0, 0))
    return pl.pallas_call(
        body,
        grid=(NP // _R,),
        in_specs=[row, row, pl.BlockSpec((_R, 2), lambda i: (i, 0)),
                  row, row, w, w, b, b],
        out_specs=[row, row],
        out_shape=[jax.ShapeDtypeStruct((NP, D), jnp.float32)] * 2,
    )(sa, sb, degT, x, hi, wxh, whh, bz, br)


def _t31(p0, p1, degT, z, hi, bh, hnext, wxz, whz, wxr, whr):
    NP, D = z.shape

    def body(p0_ref, p1_ref, deg_ref, z_ref, h_ref, bh_ref, hn_ref,
             wxz_ref, whz_ref, wxr_ref, whr_ref,
             out_ref, ya_ref, yb_ref):
        dinv = _dinv_of(deg_ref[...])
        htil = jnp.tanh(dinv * (p0_ref[...] + p1_ref[...]) + bh_ref[...])
        zb = z_ref[...]
        xb = zb * h_ref[...] + (1.0 - zb) * htil
        out_ref[...] = xb
        hb = hn_ref[...]
        ya_ref[...] = dinv * (
            jnp.dot(xb, wxz_ref[...], preferred_element_type=jnp.float32)
            + jnp.dot(hb, whz_ref[...], preferred_element_type=jnp.float32))
        yb_ref[...] = dinv * (
            jnp.dot(xb, wxr_ref[...], preferred_element_type=jnp.float32)
            + jnp.dot(hb, whr_ref[...], preferred_element_type=jnp.float32))

    row = pl.BlockSpec((_R, D), lambda i: (i, 0))
    w = pl.BlockSpec((D, D), lambda i: (0, 0))
    b = pl.BlockSpec((1, D), lambda i: (0, 0))
    return pl.pallas_call(
        body,
        grid=(NP // _R,),
        in_specs=[row, row, pl.BlockSpec((_R, 2), lambda i: (i, 0)),
                  row, row, b, row, w, w, w, w],
        out_specs=[row, row, row],
        out_shape=[jax.ShapeDtypeStruct((NP, D), jnp.float32)] * 3,
    )(p0, p1, degT, z, hi, bh, hnext, wxz, whz, wxr, whr)


def _t3(p0, p1, degT, z, hi, bh, n_out):
    NP, D = z.shape
    R = 2000
    assert n_out % R == 0

    def body(p0_ref, p1_ref, deg_ref, z_ref, h_ref, bh_ref, out_ref):
        dinv = _dinv_of(deg_ref[...])
        htil = jnp.tanh(dinv * (p0_ref[...] + p1_ref[...]) + bh_ref[...])
        zb = z_ref[...]
        out_ref[...] = zb * h_ref[...] + (1.0 - zb) * htil

    row = pl.BlockSpec((R, D), lambda i: (i, 0))
    b = pl.BlockSpec((1, D), lambda i: (0, 0))
    return pl.pallas_call(
        body,
        grid=(n_out // R,),
        in_specs=[row, row, pl.BlockSpec((R, 2), lambda i: (i, 0)),
                  row, row, b],
        out_specs=row,
        out_shape=jax.ShapeDtypeStruct((n_out, D), jnp.float32),
    )(p0, p1, degT, z, hi, bh)


def kernel(inp, edgidx, h, Wxz, Whz, Wxr, Whr, Wxh, Whh,
           bxz, bhz, bxr, bhr, bxh, bhh):
    N, D = inp.shape
    E = edgidx.shape[1]
    L = h.shape[0]
    NPAD = ((N + NW * LANES - 1) // (NW * LANES)) * (NW * LANES)
    assert NPAD % _R == 0

    assert E % _K == 0
    src = edgidx[0].astype(jnp.int32)
    dst = edgidx[1].astype(jnp.int32)
    src2d = src.reshape(E // _K, _K)
    dst2d = dst.reshape(E // _K, _K)

    pad_n = NPAD - N
    xpad = jnp.pad(inp, ((0, pad_n), (0, 0)))
    hpad = jnp.pad(h, ((0, 0), (0, pad_n), (0, 0)))

    degp = _make_deg(NPAD, E)(dst)
    degT = jnp.transpose(degp)

    prop_dual = _make_prop(NPAD, E, D, dual=True)
    prop_split = _make_prop(NPAD, E, D, dual=False)

    zeros = jnp.zeros((NPAD, D), jnp.float32)

    x = xpad
    hs = []
    ya, yb = _t1(x, hpad[0], degT, Wxz[0], Whz[0], Wxr[0], Whr[0])
    for i in range(L):
        hi = hpad[i]
        bz = (bxz[i] + bhz[i]).reshape(1, D)
        br = (bxr[i] + bhr[i]).reshape(1, D)
        bh = (bxh[i] + bhh[i]).reshape(1, D)

        sa, sb = prop_dual(ya, yb, src2d, dst2d)
        z, y2 = _t2(sa, sb, degT, x, hi, Wxh[i], Whh[i], bz, br)
        p0, p1 = prop_split(y2, zeros, src2d, dst2d)
        if i + 1 < L:
            x, ya, yb = _t31(p0, p1, degT, z, hi, bh, hpad[i + 1],
                             Wxz[i + 1], Whz[i + 1], Wxr[i + 1], Whr[i + 1])
            hs.append(x[:N])
        else:
            hs.append(_t3(p0, p1, degT, z, hi, bh, N))

    h_out = jnp.stack(hs, axis=0)
    return (h_out, h_out)


# speedup vs baseline: 32.0911x; 1.0071x over previous
import functools

import jax
import jax.numpy as jnp
from jax import lax
from jax.experimental import pallas as pl
from jax.experimental.pallas import tpu as pltpu
from jax.experimental.pallas import tpu_sc as plsc

NC = 2
NS = 16
NW = NC * NS
LANES = 16

_mesh = functools.partial(
    plsc.VectorSubcoreMesh,
    core_axis_name="c", subcore_axis_name="s",
    num_cores=NC, num_subcores=NS,
)
_sc_params = pltpu.CompilerParams(needs_layout_passes=False)


def _make_deg(NPAD, E):
    RPT = NPAD // NS
    EPT = E // NW
    assert E % NW == 0 and EPT % LANES == 0 and RPT % LANES == 0

    @functools.partial(
        pl.kernel,
        out_type=jax.ShapeDtypeStruct((NC, NPAD), jnp.float32),
        mesh=_mesh(),
        scratch_types=[
            pltpu.VMEM((NPAD,), jnp.float32),
            pltpu.VMEM((EPT,), jnp.int32),
            pltpu.VMEM((RPT,), jnp.float32),
            pltpu.VMEM((RPT,), jnp.float32),
            pltpu.VMEM_SHARED((NS, NPAD), jnp.float32),
        ],
        compiler_params=_sc_params,
    )
    def deg_kernel(dst_hbm, out_hbm, hist, dbuf, acc, tmp, hist_all):
        c = lax.axis_index("c")
        s = lax.axis_index("s")
        wid = s * NC + c

        zero16 = jnp.zeros((LANES,), jnp.float32)

        def zero_body(j, _):
            hist[pl.ds(j * LANES, LANES)] = zero16
            return 0
        lax.fori_loop(0, NPAD // LANES, zero_body, 0)

        pltpu.sync_copy(dst_hbm.at[pl.ds(wid * EPT, EPT)], dbuf)

        ones16 = jnp.ones((LANES,), jnp.float32)

        def add_body(j, _):
            idx = dbuf[pl.ds(j * LANES, LANES)]
            plsc.addupdate_scatter(hist, [idx], ones16)
            return 0
        lax.fori_loop(0, EPT // LANES, add_body, 0)

        pltpu.sync_copy(hist, hist_all.at[s])
        plsc.subcore_barrier()

        base = s * RPT
        pltpu.sync_copy(hist_all.at[0, pl.ds(base, RPT)], acc)

        def comb_body(t, _):
            pltpu.sync_copy(hist_all.at[t, pl.ds(base, RPT)], tmp)

            def add16(j, _):
                sl = pl.ds(j * LANES, LANES)
                acc[sl] = acc[sl] + tmp[sl]
                return 0
            lax.fori_loop(0, RPT // LANES, add16, 0)
            return 0
        lax.fori_loop(1, NS, comb_body, 0)

        pltpu.sync_copy(acc, out_hbm.at[c, pl.ds(base, RPT)])

    return deg_kernel


_K = 125


def _make_prop(NPAD, E, D, dual):
    K = _K
    RPT = NPAD // NS
    EPT = E // NS if dual else E // NW
    assert EPT % K == 0 and RPT % 8 == 0
    nch = EPT // K
    IB = 16 if dual else 8
    assert nch % IB == 0
    nblk = nch // IB
    assert nblk % 2 == 0


    @functools.partial(
        pl.kernel,
        out_type=[jax.ShapeDtypeStruct((NPAD, D), jnp.float32)] * 2,
        mesh=_mesh(),
        scratch_types=[
            [pltpu.VMEM((IB, K), jnp.int32)] * 2,
            [pltpu.VMEM((IB, K), jnp.int32)] * 2,
            [pltpu.VMEM((K, D), jnp.float32)] * 2,
            [pltpu.SemaphoreType.DMA] * 2,
            [pltpu.SemaphoreType.DMA] * 2,
            [pltpu.SemaphoreType.DMA] * 2,
            pltpu.VMEM_SHARED((NPAD, D), jnp.float32),
        ],
        compiler_params=_sc_params,
    )
    def prop_kernel(ya_hbm, yb_hbm, src_hbm, dst_hbm, outa_hbm, outb_hbm,
                    sblks, dblks, rows, gsem, ssem, isem, acc):
        c = lax.axis_index("c")
        s = lax.axis_index("s")
        rbase = s * RPT

        @pl.when(c == 0)
        def _():
            pltpu.sync_copy(ya_hbm.at[pl.ds(rbase, RPT)],
                            acc.at[pl.ds(rbase, RPT)])

        @pl.when(c == 1)
        def _():
            pltpu.sync_copy(yb_hbm.at[pl.ds(rbase, RPT)],
                            acc.at[pl.ds(rbase, RPT)])

        cbase = (s if dual else s * NC + c) * nch
        plsc.subcore_barrier()

        def issue_g(sblk, j, b):
            if dual:
                @pl.when(c == 0)
                def _():
                    pltpu.async_copy(ya_hbm.at[sblk.at[j]], rows[b], gsem[b])

                @pl.when(c == 1)
                def _():
                    pltpu.async_copy(yb_hbm.at[sblk.at[j]], rows[b], gsem[b])
            else:
                pltpu.async_copy(ya_hbm.at[sblk.at[j]], rows[b], gsem[b])

        def wait_g(sblk, j, b):
            pltpu.make_async_copy(ya_hbm.at[sblk.at[j]],
                                  rows[b], gsem[b]).wait()

        def issue_s(dblk, j, b):
            pltpu.async_copy(rows[b], acc.at[dblk.at[j]], ssem[b], add=True)

        def wait_s(dblk, j, b):
            pltpu.make_async_copy(rows[b], acc.at[dblk.at[j]],
                                  ssem[b]).wait()

        def refill(blk, hb):
            row0 = cbase + blk * IB
            pltpu.async_copy(src_hbm.at[pl.ds(row0, IB)], sblks[hb], isem[hb])
            pltpu.async_copy(dst_hbm.at[pl.ds(row0, IB)], dblks[hb], isem[hb])

        def wait_refill(hb):
            pltpu.make_async_copy(src_hbm.at[pl.ds(0, IB)],
                                  sblks[hb], isem[hb]).wait()
            pltpu.make_async_copy(dst_hbm.at[pl.ds(0, IB)],
                                  dblks[hb], isem[hb]).wait()

        def run_block(hb, last_tail):
            sblk, dblk = sblks[hb], dblks[hb]

            def pair_body(q, _):
                j = 2 * q
                wait_g(sblk, j, 0)
                issue_s(dblk, j, 0)

                @pl.when(q > 0)
                def _():
                    wait_s(dblk, j - 1, 1)
                issue_g(sblk, j + 1, 1)

                wait_g(sblk, j + 1, 1)
                issue_s(dblk, j + 1, 1)
                wait_s(dblk, j, 0)

                @pl.when(q < IB // 2 - 1)
                def _():
                    issue_g(sblk, j + 2, 0)
                return 0
            lax.fori_loop(0, IB // 2, pair_body, 0)
            wait_s(dblk, IB - 1, 1)
            last_tail()

        pltpu.sync_copy(src_hbm.at[pl.ds(cbase, IB)], sblks[0])
        pltpu.sync_copy(dst_hbm.at[pl.ds(cbase, IB)], dblks[0])
        refill(1, 1)
        issue_g(sblks[0], 0, 0)

        def sb_body(sb, _):
            blk = 2 * sb

            def tail_a():
                wait_refill(1)
                issue_g(sblks[1], 0, 0)

                @pl.when(sb < nblk // 2 - 1)
                def _():
                    refill(blk + 2, 0)
            run_block(0, tail_a)

            def tail_b():
                @pl.when(sb < nblk // 2 - 1)
                def _():
                    wait_refill(0)
                    issue_g(sblks[0], 0, 0)
                    refill(blk + 3, 1)
            run_block(1, tail_b)
            return 0
        lax.fori_loop(0, nblk // 2, sb_body, 0)

        plsc.subcore_barrier()

        @pl.when(c == 0)
        def _():
            pltpu.sync_copy(acc.at[pl.ds(rbase, RPT)],
                            outa_hbm.at[pl.ds(rbase, RPT)])

        @pl.when(c == 1)
        def _():
            pltpu.sync_copy(acc.at[pl.ds(rbase, RPT)],
                            outb_hbm.at[pl.ds(rbase, RPT)])

    return prop_kernel


_R = 2048


def _dinv_of(deg_blk):
    return lax.rsqrt(deg_blk[:, 0:1] + deg_blk[:, 1:2] + 1.0)


def _t1(x, hi, degT, wxz, whz, wxr, whr):
    NP, D = x.shape

    def body(x_ref, h_ref, deg_ref, wxz_ref, whz_ref, wxr_ref, whr_ref,
             ya_ref, yb_ref):
        dinv = _dinv_of(deg_ref[...])
        xb = x_ref[...]
        hb = h_ref[...]
        ya_ref[...] = dinv * (
            jnp.dot(xb, wxz_ref[...], preferred_element_type=jnp.float32)
            + jnp.dot(hb, whz_ref[...], preferred_element_type=jnp.float32))
        yb_ref[...] = dinv * (
            jnp.dot(xb, wxr_ref[...], preferred_element_type=jnp.float32)
            + jnp.dot(hb, whr_ref[...], preferred_element_type=jnp.float32))

    row = pl.BlockSpec((_R, D), lambda i: (i, 0))
    w = pl.BlockSpec((D, D), lambda i: (0, 0))
    return pl.pallas_call(
        body,
        grid=(NP // _R,),
        in_specs=[row, row, pl.BlockSpec((_R, 2), lambda i: (i, 0)),
                  w, w, w, w],
        out_specs=[row, row],
        out_shape=[jax.ShapeDtypeStruct((NP, D), jnp.float32)] * 2,
    )(x, hi, degT, wxz, whz, wxr, whr)


def _t2(sa, sb, degT, x, hi, wxh, whh, bz, br):
    NP, D = x.shape

    def body(sa_ref, sb_ref, deg_ref, x_ref, h_ref, wxh_ref, whh_ref,
             bz_ref, br_ref, z_ref, y2_ref):
        dinv = _dinv_of(deg_ref[...])
        z = jax.nn.sigmoid(dinv * sa_ref[...] + bz_ref[...])
        r = jax.nn.sigmoid(dinv * sb_ref[...] + br_ref[...])
        y2 = dinv * (
            jnp.dot(x_ref[...], wxh_ref[...],
                    preferred_element_type=jnp.float32)
            + jnp.dot(r * h_ref[...], whh_ref[...],
                      preferred_element_type=jnp.float32))
        z_ref[...] = z
        y2_ref[...] = y2

    row = pl.BlockSpec((_R, D), lambda i: (i, 0))
    w = pl.BlockSpec((D, D), lambda i: (0, 0))
    b = pl.BlockSpec((1, D), lambda i: (0, 0))
    return pl.pallas_call(
        body,
        grid=(NP // _R,),
        in_specs=[row, row, pl.BlockSpec((_R, 2), lambda i: (i, 0)),
                  row, row, w, w, b, b],
        out_specs=[row, row],
        out_shape=[jax.ShapeDtypeStruct((NP, D), jnp.float32)] * 2,
    )(sa, sb, degT, x, hi, wxh, whh, bz, br)


def _t31(p0, p1, degT, z, hi, bh, hnext, wxz, whz, wxr, whr):
    NP, D = z.shape

    def body(p0_ref, p1_ref, deg_ref, z_ref, h_ref, bh_ref, hn_ref,
             wxz_ref, whz_ref, wxr_ref, whr_ref,
             out_ref, ya_ref, yb_ref):
        dinv = _dinv_of(deg_ref[...])
        htil = jnp.tanh(dinv * (p0_ref[...] + p1_ref[...]) + bh_ref[...])
        zb = z_ref[...]
        xb = zb * h_ref[...] + (1.0 - zb) * htil
        out_ref[...] = xb
        hb = hn_ref[...]
        ya_ref[...] = dinv * (
            jnp.dot(xb, wxz_ref[...], preferred_element_type=jnp.float32)
            + jnp.dot(hb, whz_ref[...], preferred_element_type=jnp.float32))
        yb_ref[...] = dinv * (
            jnp.dot(xb, wxr_ref[...], preferred_element_type=jnp.float32)
            + jnp.dot(hb, whr_ref[...], preferred_element_type=jnp.float32))

    row = pl.BlockSpec((_R, D), lambda i: (i, 0))
    w = pl.BlockSpec((D, D), lambda i: (0, 0))
    b = pl.BlockSpec((1, D), lambda i: (0, 0))
    return pl.pallas_call(
        body,
        grid=(NP // _R,),
        in_specs=[row, row, pl.BlockSpec((_R, 2), lambda i: (i, 0)),
                  row, row, b, row, w, w, w, w],
        out_specs=[row, row, row],
        out_shape=[jax.ShapeDtypeStruct((NP, D), jnp.float32)] * 3,
    )(p0, p1, degT, z, hi, bh, hnext, wxz, whz, wxr, whr)


def _t3(p0, p1, degT, z, hi, bh, n_out):
    NP, D = z.shape
    R = 2000
    assert n_out % R == 0

    def body(p0_ref, p1_ref, deg_ref, z_ref, h_ref, bh_ref, out_ref):
        dinv = _dinv_of(deg_ref[...])
        htil = jnp.tanh(dinv * (p0_ref[...] + p1_ref[...]) + bh_ref[...])
        zb = z_ref[...]
        out_ref[...] = zb * h_ref[...] + (1.0 - zb) * htil

    row = pl.BlockSpec((R, D), lambda i: (i, 0))
    b = pl.BlockSpec((1, D), lambda i: (0, 0))
    return pl.pallas_call(
        body,
        grid=(n_out // R,),
        in_specs=[row, row, pl.BlockSpec((R, 2), lambda i: (i, 0)),
                  row, row, b],
        out_specs=row,
        out_shape=jax.ShapeDtypeStruct((n_out, D), jnp.float32),
    )(p0, p1, degT, z, hi, bh)


def kernel(inp, edgidx, h, Wxz, Whz, Wxr, Whr, Wxh, Whh,
           bxz, bhz, bxr, bhr, bxh, bhh):
    N, D = inp.shape
    E = edgidx.shape[1]
    L = h.shape[0]
    NPAD = ((N + NW * LANES - 1) // (NW * LANES)) * (NW * LANES)
    assert NPAD % _R == 0

    assert E % _K == 0
    src = edgidx[0].astype(jnp.int32)
    dst = edgidx[1].astype(jnp.int32)
    src2d = src.reshape(E // _K, _K)
    dst2d = dst.reshape(E // _K, _K)

    pad_n = NPAD - N
    xpad = jnp.pad(inp, ((0, pad_n), (0, 0)))
    hpad = jnp.pad(h, ((0, 0), (0, pad_n), (0, 0)))

    degp = _make_deg(NPAD, E)(dst)
    degT = jnp.transpose(degp)

    prop_dual = _make_prop(NPAD, E, D, dual=True)
    prop_split = _make_prop(NPAD, E, D, dual=False)

    zeros = jnp.zeros((NPAD, D), jnp.float32)

    x = xpad
    hs = []
    ya, yb = _t1(x, hpad[0], degT, Wxz[0], Whz[0], Wxr[0], Whr[0])
    for i in range(L):
        hi = hpad[i]
        bz = (bxz[i] + bhz[i]).reshape(1, D)
        br = (bxr[i] + bhr[i]).reshape(1, D)
        bh = (bxh[i] + bhh[i]).reshape(1, D)

        sa, sb = prop_dual(ya, yb, src2d, dst2d)
        z, y2 = _t2(sa, sb, degT, x, hi, Wxh[i], Whh[i], bz, br)
        p0, p1 = prop_split(y2, zeros, src2d, dst2d)
        if i + 1 < L:
            x, ya, yb = _t31(p0, p1, degT, z, hi, bh, hpad[i + 1],
                             Wxz[i + 1], Whz[i + 1], Wxr[i + 1], Whr[i + 1])
            hs.append(x[:N])
        else:
            hs.append(_t3(p0, p1, degT, z, hi, bh, N))

    h_out = jnp.stack(hs, axis=0)
    return (h_out, h_out)


# speedup vs baseline: 32.3737x; 1.0088x over previous
import functools

import jax
import jax.numpy as jnp
from jax import lax
from jax.experimental import pallas as pl
from jax.experimental.pallas import tpu as pltpu
from jax.experimental.pallas import tpu_sc as plsc

NC = 2
NS = 16
NW = NC * NS
LANES = 16

_mesh = functools.partial(
    plsc.VectorSubcoreMesh,
    core_axis_name="c", subcore_axis_name="s",
    num_cores=NC, num_subcores=NS,
)
_sc_params = pltpu.CompilerParams(needs_layout_passes=False)


def _make_deg(NPAD, E):
    RPT = NPAD // NS
    EPT = E // NW
    assert E % NW == 0 and EPT % LANES == 0 and RPT % LANES == 0

    @functools.partial(
        pl.kernel,
        out_type=jax.ShapeDtypeStruct((NC, NPAD), jnp.float32),
        mesh=_mesh(),
        scratch_types=[
            pltpu.VMEM((NPAD,), jnp.float32),
            pltpu.VMEM((EPT,), jnp.int32),
            pltpu.VMEM((RPT,), jnp.float32),
            pltpu.VMEM((RPT,), jnp.float32),
            pltpu.VMEM_SHARED((NS, NPAD), jnp.float32),
        ],
        compiler_params=_sc_params,
    )
    def deg_kernel(dst_hbm, out_hbm, hist, dbuf, acc, tmp, hist_all):
        c = lax.axis_index("c")
        s = lax.axis_index("s")
        wid = s * NC + c

        zero16 = jnp.zeros((LANES,), jnp.float32)

        def zero_body(j, _):
            hist[pl.ds(j * LANES, LANES)] = zero16
            return 0
        lax.fori_loop(0, NPAD // LANES, zero_body, 0)

        pltpu.sync_copy(dst_hbm.at[pl.ds(wid * EPT, EPT)], dbuf)

        ones16 = jnp.ones((LANES,), jnp.float32)

        def add_body(j, _):
            idx = dbuf[pl.ds(j * LANES, LANES)]
            plsc.addupdate_scatter(hist, [idx], ones16)
            return 0
        lax.fori_loop(0, EPT // LANES, add_body, 0)

        pltpu.sync_copy(hist, hist_all.at[s])
        plsc.subcore_barrier()

        base = s * RPT
        pltpu.sync_copy(hist_all.at[0, pl.ds(base, RPT)], acc)

        def comb_body(t, _):
            pltpu.sync_copy(hist_all.at[t, pl.ds(base, RPT)], tmp)

            def add16(j, _):
                sl = pl.ds(j * LANES, LANES)
                acc[sl] = acc[sl] + tmp[sl]
                return 0
            lax.fori_loop(0, RPT // LANES, add16, 0)
            return 0
        lax.fori_loop(1, NS, comb_body, 0)

        pltpu.sync_copy(acc, out_hbm.at[c, pl.ds(base, RPT)])

    return deg_kernel


_K = 125


def _make_prop(NPAD, E, D, dual):
    K = _K
    RPT = NPAD // NS
    EPT = E // NS if dual else E // NW
    assert EPT % K == 0 and RPT % 8 == 0
    nch = EPT // K
    IB = 16 if dual else 8
    assert nch % IB == 0
    nblk = nch // IB
    assert nblk % 2 == 0


    @functools.partial(
        pl.kernel,
        out_type=[jax.ShapeDtypeStruct((NPAD, D), jnp.float32)] * 2,
        mesh=_mesh(),
        scratch_types=[
            [pltpu.VMEM((IB, K), jnp.int32)] * 2,
            [pltpu.VMEM((IB, K), jnp.int32)] * 2,
            [pltpu.VMEM((K, D), jnp.float32)] * 2,
            [pltpu.SemaphoreType.DMA] * 2,
            [pltpu.SemaphoreType.DMA] * 2,
            [pltpu.SemaphoreType.DMA] * 2,
            pltpu.VMEM_SHARED((NPAD, D), jnp.float32),
        ],
        compiler_params=_sc_params,
    )
    def prop_kernel(ya_hbm, yb_hbm, src_hbm, dst_hbm, outa_hbm, outb_hbm,
                    sblks, dblks, rows, gsem, ssem, isem, acc):
        c = lax.axis_index("c")
        s = lax.axis_index("s")
        rbase = s * RPT

        @pl.when(c == 0)
        def _():
            pltpu.sync_copy(ya_hbm.at[pl.ds(rbase, RPT)],
                            acc.at[pl.ds(rbase, RPT)])

        @pl.when(c == 1)
        def _():
            pltpu.sync_copy(yb_hbm.at[pl.ds(rbase, RPT)],
                            acc.at[pl.ds(rbase, RPT)])

        cbase = (s if dual else s * NC + c) * nch
        plsc.subcore_barrier()

        def issue_g(sblk, j, b):
            if dual:
                @pl.when(c == 0)
                def _():
                    pltpu.async_copy(ya_hbm.at[sblk.at[j]], rows[b], gsem[b])

                @pl.when(c == 1)
                def _():
                    pltpu.async_copy(yb_hbm.at[sblk.at[j]], rows[b], gsem[b])
            else:
                pltpu.async_copy(ya_hbm.at[sblk.at[j]], rows[b], gsem[b])

        def wait_g(sblk, j, b):
            pltpu.make_async_copy(ya_hbm.at[sblk.at[j]],
                                  rows[b], gsem[b]).wait()

        def issue_s(dblk, j, b):
            pltpu.async_copy(rows[b], acc.at[dblk.at[j]], ssem[b], add=True)

        def wait_s(dblk, j, b):
            pltpu.make_async_copy(rows[b], acc.at[dblk.at[j]],
                                  ssem[b]).wait()

        def refill(blk, hb):
            row0 = cbase + blk * IB
            pltpu.async_copy(src_hbm.at[pl.ds(row0, IB)], sblks[hb], isem[hb])
            pltpu.async_copy(dst_hbm.at[pl.ds(row0, IB)], dblks[hb], isem[hb])

        def wait_refill(hb):
            pltpu.make_async_copy(src_hbm.at[pl.ds(0, IB)],
                                  sblks[hb], isem[hb]).wait()
            pltpu.make_async_copy(dst_hbm.at[pl.ds(0, IB)],
                                  dblks[hb], isem[hb]).wait()

        def run_block(hb, last_tail):
            sblk, dblk = sblks[hb], dblks[hb]

            def pair_body(q, _):
                j = 2 * q
                wait_g(sblk, j, 0)
                issue_s(dblk, j, 0)

                @pl.when(q > 0)
                def _():
                    wait_s(dblk, j - 1, 1)
                issue_g(sblk, j + 1, 1)

                wait_g(sblk, j + 1, 1)
                issue_s(dblk, j + 1, 1)
                wait_s(dblk, j, 0)

                @pl.when(q < IB // 2 - 1)
                def _():
                    issue_g(sblk, j + 2, 0)
                return 0
            lax.fori_loop(0, IB // 2, pair_body, 0)
            wait_s(dblk, IB - 1, 1)
            last_tail()

        pltpu.sync_copy(src_hbm.at[pl.ds(cbase, IB)], sblks[0])
        pltpu.sync_copy(dst_hbm.at[pl.ds(cbase, IB)], dblks[0])
        refill(1, 1)
        issue_g(sblks[0], 0, 0)

        def sb_body(sb, _):
            blk = 2 * sb

            def tail_a():
                wait_refill(1)
                issue_g(sblks[1], 0, 0)

                @pl.when(sb < nblk // 2 - 1)
                def _():
                    refill(blk + 2, 0)
            run_block(0, tail_a)

            def tail_b():
                @pl.when(sb < nblk // 2 - 1)
                def _():
                    wait_refill(0)
                    issue_g(sblks[0], 0, 0)
                    refill(blk + 3, 1)
            run_block(1, tail_b)
            return 0
        lax.fori_loop(0, nblk // 2, sb_body, 0)

        plsc.subcore_barrier()

        @pl.when(c == 0)
        def _():
            pltpu.sync_copy(acc.at[pl.ds(rbase, RPT)],
                            outa_hbm.at[pl.ds(rbase, RPT)])

        @pl.when(c == 1)
        def _():
            pltpu.sync_copy(acc.at[pl.ds(rbase, RPT)],
                            outb_hbm.at[pl.ds(rbase, RPT)])

    return prop_kernel


_R = 2000


def _dinv_of(deg_blk):
    return lax.rsqrt(deg_blk[:, 0:1] + deg_blk[:, 1:2] + 1.0)


_row = pl.BlockSpec((_R, 128), lambda i: (i, 0))
_deg = pl.BlockSpec((_R, 2), lambda i: (i, 0))
_wb = pl.BlockSpec((128, 128), lambda i: (0, 0))
_bb = pl.BlockSpec((1, 128), lambda i: (0, 0))


def _t1(NPAD, x, hi, degT, wxz, whz, wxr, whr):
    N, D = x.shape

    def body(x_ref, h_ref, deg_ref, wxz_ref, whz_ref, wxr_ref, whr_ref,
             ya_ref, yb_ref):
        dinv = _dinv_of(deg_ref[...])
        xb = x_ref[...]
        hb = h_ref[...]
        ya_ref[...] = dinv * (
            jnp.dot(xb, wxz_ref[...], preferred_element_type=jnp.float32)
            + jnp.dot(hb, whz_ref[...], preferred_element_type=jnp.float32))
        yb_ref[...] = dinv * (
            jnp.dot(xb, wxr_ref[...], preferred_element_type=jnp.float32)
            + jnp.dot(hb, whr_ref[...], preferred_element_type=jnp.float32))

    return pl.pallas_call(
        body,
        grid=(N // _R,),
        in_specs=[_row, _row, _deg, _wb, _wb, _wb, _wb],
        out_specs=[_row, _row],
        out_shape=[jax.ShapeDtypeStruct((NPAD, D), jnp.float32)] * 2,
    )(x, hi, degT, wxz, whz, wxr, whr)


def _t2(NPAD, sa, sb, degT, x, hi, wxh, whh, bz, br):
    N, D = x.shape

    def body(sa_ref, sb_ref, deg_ref, x_ref, h_ref, wxh_ref, whh_ref,
             bz_ref, br_ref, z_ref, y2_ref):
        dinv = _dinv_of(deg_ref[...])
        z = jax.nn.sigmoid(dinv * sa_ref[...] + bz_ref[...])
        r = jax.nn.sigmoid(dinv * sb_ref[...] + br_ref[...])
        y2 = dinv * (
            jnp.dot(x_ref[...], wxh_ref[...],
                    preferred_element_type=jnp.float32)
            + jnp.dot(r * h_ref[...], whh_ref[...],
                      preferred_element_type=jnp.float32))
        z_ref[...] = z
        y2_ref[...] = y2

    return pl.pallas_call(
        body,
        grid=(N // _R,),
        in_specs=[_row, _row, _deg, _row, _row, _wb, _wb, _bb, _bb],
        out_specs=[_row, _row],
        out_shape=[jax.ShapeDtypeStruct((N, D), jnp.float32),
                   jax.ShapeDtypeStruct((NPAD, D), jnp.float32)],
    )(sa, sb, degT, x, hi, wxh, whh, bz, br)


def _t31(NPAD, p0, p1, degT, z, hi, bh, hnext, wxz, whz, wxr, whr):
    N, D = z.shape

    def body(p0_ref, p1_ref, deg_ref, z_ref, h_ref, bh_ref, hn_ref,
             wxz_ref, whz_ref, wxr_ref, whr_ref,
             out_ref, ya_ref, yb_ref):
        dinv = _dinv_of(deg_ref[...])
        htil = jnp.tanh(dinv * (p0_ref[...] + p1_ref[...]) + bh_ref[...])
        zb = z_ref[...]
        xb = zb * h_ref[...] + (1.0 - zb) * htil
        out_ref[...] = xb
        hb = hn_ref[...]
        ya_ref[...] = dinv * (
            jnp.dot(xb, wxz_ref[...], preferred_element_type=jnp.float32)
            + jnp.dot(hb, whz_ref[...], preferred_element_type=jnp.float32))
        yb_ref[...] = dinv * (
            jnp.dot(xb, wxr_ref[...], preferred_element_type=jnp.float32)
            + jnp.dot(hb, whr_ref[...], preferred_element_type=jnp.float32))

    return pl.pallas_call(
        body,
        grid=(N // _R,),
        in_specs=[_row, _row, _deg, _row, _row, _bb, _row,
                  _wb, _wb, _wb, _wb],
        out_specs=[_row, _row, _row],
        out_shape=[jax.ShapeDtypeStruct((N, D), jnp.float32),
                   jax.ShapeDtypeStruct((NPAD, D), jnp.float32),
                   jax.ShapeDtypeStruct((NPAD, D), jnp.float32)],
    )(p0, p1, degT, z, hi, bh, hnext, wxz, whz, wxr, whr)


def _t3(p0, p1, degT, z, hi, bh):
    N, D = z.shape

    def body(p0_ref, p1_ref, deg_ref, z_ref, h_ref, bh_ref, out_ref):
        dinv = _dinv_of(deg_ref[...])
        htil = jnp.tanh(dinv * (p0_ref[...] + p1_ref[...]) + bh_ref[...])
        zb = z_ref[...]
        out_ref[...] = zb * h_ref[...] + (1.0 - zb) * htil

    return pl.pallas_call(
        body,
        grid=(N // _R,),
        in_specs=[_row, _row, _deg, _row, _row, _bb],
        out_specs=_row,
        out_shape=jax.ShapeDtypeStruct((N, D), jnp.float32),
    )(p0, p1, degT, z, hi, bh)


def kernel(inp, edgidx, h, Wxz, Whz, Wxr, Whr, Wxh, Whh,
           bxz, bhz, bxr, bhr, bxh, bhh):
    N, D = inp.shape
    E = edgidx.shape[1]
    L = h.shape[0]
    NPAD = ((N + NW * LANES - 1) // (NW * LANES)) * (NW * LANES)
    assert N % _R == 0

    assert E % _K == 0
    src = edgidx[0].astype(jnp.int32)
    dst = edgidx[1].astype(jnp.int32)
    src2d = src.reshape(E // _K, _K)
    dst2d = dst.reshape(E // _K, _K)

    degp = _make_deg(NPAD, E)(dst)
    degT = jnp.transpose(degp)

    prop_dual = _make_prop(NPAD, E, D, dual=True)
    prop_split = _make_prop(NPAD, E, D, dual=False)

    zeros = jnp.zeros((NPAD, D), jnp.float32)

    x = inp
    hs = []
    ya, yb = _t1(NPAD, x, h[0], degT, Wxz[0], Whz[0], Wxr[0], Whr[0])
    for i in range(L):
        hi = h[i]
        bz = (bxz[i] + bhz[i]).reshape(1, D)
        br = (bxr[i] + bhr[i]).reshape(1, D)
        bh = (bxh[i] + bhh[i]).reshape(1, D)

        sa, sb = prop_dual(ya, yb, src2d, dst2d)
        z, y2 = _t2(NPAD, sa, sb, degT, x, hi, Wxh[i], Whh[i], bz, br)
        p0, p1 = prop_split(y2, zeros, src2d, dst2d)
        if i + 1 < L:
            x, ya, yb = _t31(NPAD, p0, p1, degT, z, hi, bh, h[i + 1],
                             Wxz[i + 1], Whz[i + 1], Wxr[i + 1], Whr[i + 1])
            hs.append(x)
        else:
            hs.append(_t3(p0, p1, degT, z, hi, bh))

    h_out = jnp.stack(hs, axis=0)
    return (h_out, h_out)
